# baseline (speedup 1.0000x reference)
import functools

import jax
import jax.numpy as jnp
from jax import lax
from jax.experimental import pallas as pl
from jax.experimental.pallas import tpu as pltpu
from jax.experimental.pallas import tpu_sc as plsc

N = 160000
D_IN = 256
D_HID = 512
N_BINS = 10000


_ROWS_BLK = 10000


def _mlp_body(x_ref, w1_ref, b1_ref, w2_ref, b2_ref, o_ref):
    h = jnp.dot(x_ref[...], w1_ref[...], preferred_element_type=jnp.float32)
    h = jnp.maximum(h + b1_ref[...], 0.0)
    y = jnp.dot(h, w2_ref[...], preferred_element_type=jnp.float32)
    o_ref[...] = y + b2_ref[...]


def _mlp(x, W1, b1, W2, b2):
    return pl.pallas_call(
        _mlp_body,
        grid=(N // _ROWS_BLK,),
        in_specs=[
            pl.BlockSpec((_ROWS_BLK, D_IN), lambda i: (i, 0)),
            pl.BlockSpec((D_IN, D_HID), lambda i: (0, 0)),
            pl.BlockSpec((1, D_HID), lambda i: (0, 0)),
            pl.BlockSpec((D_HID, D_IN), lambda i: (0, 0)),
            pl.BlockSpec((1, D_IN), lambda i: (0, 0)),
        ],
        out_specs=pl.BlockSpec((_ROWS_BLK, D_IN), lambda i: (i, 0)),
        out_shape=jax.ShapeDtypeStruct((N, D_IN), jnp.float32),
    )(x, W1, b1.reshape(1, D_HID), W2, b2.reshape(1, D_IN))


_NC, _NS = 2, 16
_HALF = D_IN // _NC
_ROWS_PER_SUB = N // _NS
_CH = 80
_NCHUNK = _ROWS_PER_SUB // _CH
_NBUF = 3
_ZERO_CH = 16
_N_ZERO_CHUNKS = N_BINS // _ZERO_CH
_EXP_CH = 40
_N_EXP_CHUNKS = N_BINS // _EXP_CH


def _segsum(items, idx3d):
    mesh = plsc.VectorSubcoreMesh(
        core_axis_name="c", subcore_axis_name="s",
        num_cores=_NC, num_subcores=_NS,
    )

    @functools.partial(
        pl.kernel,
        out_type=jax.ShapeDtypeStruct((N_BINS, D_IN), jnp.float32),
        mesh=mesh,
        scratch_types=[
            pltpu.VMEM((_NCHUNK, _CH), jnp.int32),
            pltpu.VMEM((_CH, _HALF), jnp.float32),
            pltpu.VMEM((_CH, _HALF), jnp.float32),
            pltpu.VMEM((_CH, _HALF), jnp.float32),
            pltpu.VMEM((_ZERO_CH, _HALF), jnp.float32),
            pltpu.VMEM_SHARED((N_BINS, _HALF), jnp.float32),
            pltpu.SemaphoreType.DMA,
            pltpu.SemaphoreType.DMA,
            pltpu.SemaphoreType.DMA,
            pltpu.SemaphoreType.DMA,
            pltpu.SemaphoreType.DMA,
            pltpu.SemaphoreType.DMA,
        ],
    )
    def k(items_hbm, idx_hbm, out_hbm,
          idx_v, rows0, rows1, rows2, stage_v, acc_sh,
          ls0, ls1, ls2, ss0, ss1, ss2):
        c = lax.axis_index("c")
        s = lax.axis_index("s")
        col0 = c * _HALF
        row_base = s * _ROWS_PER_SUB
        bufs = (rows0, rows1, rows2)
        lsems = (ls0, ls1, ls2)
        ssems = (ss0, ss1, ss2)

        pltpu.sync_copy(idx_hbm.at[s], idx_v)

        zero = jnp.zeros((16,), jnp.float32)

        def zst(i, carry):
            for j in range(_HALF // 16):
                stage_v[i, pl.ds(j * 16, 16)] = zero
            return carry

        lax.fori_loop(0, _ZERO_CH, zst, 0)

        n_z = (_N_ZERO_CHUNKS - s + _NS - 1) // _NS

        def zacc(t, carry):
            r0 = (s + t * _NS) * _ZERO_CH
            pltpu.sync_copy(stage_v, acc_sh.at[pl.ds(r0, _ZERO_CH)])
            return carry

        lax.fori_loop(0, n_z, zacc, 0)
        plsc.subcore_barrier()

        def start_load(chunk, b):
            row0 = row_base + chunk * _CH
            pltpu.async_copy(
                items_hbm.at[pl.ds(row0, _CH), pl.ds(col0, _HALF)],
                bufs[b], lsems[b])

        def wait_load(b):
            pltpu.make_async_copy(
                items_hbm.at[pl.ds(row_base, _CH), pl.ds(col0, _HALF)],
                bufs[b], lsems[b]).wait()

        def start_scat(chunk, b):
            pltpu.async_copy(
                bufs[b], acc_sh.at[idx_v.at[chunk]], ssems[b], add=True)

        def wait_scat(b):
            pltpu.make_async_copy(
                bufs[b], acc_sh.at[idx_v.at[0]], ssems[b]).wait()

        start_load(0, 0)
        start_load(1, 1)

        def triple(i, carry):
            for r in range(_NBUF):
                j = _NBUF * i + r
                b = r
                wait_load(b)
                start_scat(j, b)
                jn = j + 2
                bn = (r + 2) % _NBUF
                if r == 0:
                    @pl.when(i > 0)
                    def _():
                        wait_scat(bn)
                        start_load(jn, bn)

                    @pl.when(i == 0)
                    def _():
                        start_load(jn, bn)
                else:
                    wait_scat(bn)
                    start_load(jn, bn)

            return carry

        lax.fori_loop(0, _NCHUNK // _NBUF, triple, 0)
        wait_load(0)
        start_scat(_NCHUNK - 2, 0)
        wait_load(1)
        start_scat(_NCHUNK - 1, 1)
        wait_scat(0)
        wait_scat(1)
        wait_scat(2)
        plsc.subcore_barrier()

        n_t = (_N_EXP_CHUNKS - s + _NS - 1) // _NS

        def export(t, carry):
            r0 = (s + t * _NS) * _EXP_CH
            pltpu.sync_copy(
                acc_sh.at[pl.ds(r0, _EXP_CH)],
                out_hbm.at[pl.ds(r0, _EXP_CH), pl.ds(col0, _HALF)])
            return carry

        lax.fori_loop(0, n_t, export, 0)

    return k(items, idx3d)


def kernel(x, idxs, n_bins, W1, b1, W2, b2):
    idx32 = jnp.minimum(idxs, N_BINS - 1).astype(jnp.int32)
    idx3d = idx32.reshape(_NS, _NCHUNK, _CH)
    items = _mlp(x, W1, b1, W2, b2)
    return _segsum(items, idx3d)


# speedup vs baseline: 1.0286x; 1.0286x over previous
import functools

import jax
import jax.numpy as jnp
from jax import lax
from jax.experimental import pallas as pl
from jax.experimental.pallas import tpu as pltpu
from jax.experimental.pallas import tpu_sc as plsc

N = 160000
D_IN = 256
D_HID = 512
N_BINS = 10000


_ROWS_BLK = 10000


def _mlp_body(x_ref, w1_ref, b1_ref, w2_ref, b2_ref, o_ref):
    h = jnp.dot(x_ref[...], w1_ref[...], preferred_element_type=jnp.float32)
    h = jnp.maximum(h + b1_ref[...], 0.0)
    y = jnp.dot(h, w2_ref[...], preferred_element_type=jnp.float32)
    o_ref[...] = y + b2_ref[...]


def _mlp(x, W1, b1, W2, b2):
    return pl.pallas_call(
        _mlp_body,
        grid=(N // _ROWS_BLK,),
        in_specs=[
            pl.BlockSpec((_ROWS_BLK, D_IN), lambda i: (i, 0)),
            pl.BlockSpec((D_IN, D_HID), lambda i: (0, 0)),
            pl.BlockSpec((1, D_HID), lambda i: (0, 0)),
            pl.BlockSpec((D_HID, D_IN), lambda i: (0, 0)),
            pl.BlockSpec((1, D_IN), lambda i: (0, 0)),
        ],
        out_specs=pl.BlockSpec((_ROWS_BLK, D_IN), lambda i: (i, 0)),
        out_shape=jax.ShapeDtypeStruct((N, D_IN), jnp.float32),
    )(x, W1, b1.reshape(1, D_HID), W2, b2.reshape(1, D_IN))


_NC, _NS = 2, 16
_HALF = D_IN // _NC
_ROWS_PER_SUB = N // _NS
_CH = 128
_NCHUNK = _ROWS_PER_SUB // _CH
_TAIL = _ROWS_PER_SUB - _NCHUNK * _CH
_EXP_CH = 40
_N_EXP_CHUNKS = N_BINS // _EXP_CH


def _segsum(items, idx3d, idx_tail):
    mesh = plsc.VectorSubcoreMesh(
        core_axis_name="c", subcore_axis_name="s",
        num_cores=_NC, num_subcores=_NS,
    )

    @functools.partial(
        pl.kernel,
        out_type=jax.ShapeDtypeStruct((N_BINS, D_IN), jnp.float32),
        mesh=mesh,
        scratch_types=[
            pltpu.VMEM((_NCHUNK, _CH), jnp.int32),
            pltpu.VMEM((_TAIL,), jnp.int32),
            pltpu.VMEM((_CH, _HALF), jnp.float32),
            pltpu.VMEM((_CH, _HALF), jnp.float32),
            pltpu.VMEM((_TAIL, _HALF), jnp.float32),
            pltpu.VMEM((_EXP_CH, _HALF), jnp.float32),
            pltpu.VMEM_SHARED((N_BINS, _HALF), jnp.float32),
            pltpu.SemaphoreType.DMA,
            pltpu.SemaphoreType.DMA,
        ],
    )
    def k(items_hbm, idx_hbm, idxt_hbm, out_hbm,
          idx_v, idxt_v, rows0, rows1, rowst, stage_v, acc_sh, sem0, sem1):
        c = lax.axis_index("c")
        s = lax.axis_index("s")
        col0 = c * _HALF
        row_base = s * _ROWS_PER_SUB

        pltpu.sync_copy(idx_hbm.at[s], idx_v)
        pltpu.sync_copy(idxt_hbm.at[s], idxt_v)

        zero = jnp.zeros((16,), jnp.float32)

        def zst(i, carry):
            for j in range(_HALF // 16):
                stage_v[i, pl.ds(j * 16, 16)] = zero
            return carry

        lax.fori_loop(0, _EXP_CH, zst, 0)

        n_t = (_N_EXP_CHUNKS - s + _NS - 1) // _NS

        def zacc(t, carry):
            r0 = (s + t * _NS) * _EXP_CH
            pltpu.sync_copy(stage_v, acc_sh.at[pl.ds(r0, _EXP_CH)])
            return carry

        lax.fori_loop(0, n_t, zacc, 0)
        plsc.subcore_barrier()

        def start(chunk, buf, sem):
            row0 = row_base + chunk * _CH
            pltpu.async_copy(
                items_hbm.at[pl.ds(row0, _CH), pl.ds(col0, _HALF)], buf, sem)

        def wait(buf, sem):
            pltpu.make_async_copy(
                items_hbm.at[pl.ds(row_base, _CH), pl.ds(col0, _HALF)],
                buf, sem).wait()

        def scat(chunk, buf):
            pltpu.sync_copy(buf, acc_sh.at[idx_v.at[chunk]], add=True)

        start(0, rows0, sem0)

        def pair(i, carry):
            c0 = 2 * i
            c1 = c0 + 1
            start(c1, rows1, sem1)
            wait(rows0, sem0)
            scat(c0, rows0)

            @pl.when(c1 + 1 < _NCHUNK)
            def _():
                start(c1 + 1, rows0, sem0)

            wait(rows1, sem1)
            scat(c1, rows1)
            return carry

        lax.fori_loop(0, _NCHUNK // 2, pair, 0)
        pltpu.sync_copy(
            items_hbm.at[pl.ds(row_base + _NCHUNK * _CH, _TAIL),
                         pl.ds(col0, _HALF)], rowst)
        pltpu.sync_copy(rowst, acc_sh.at[idxt_v], add=True)
        plsc.subcore_barrier()

        def export(t, carry):
            r0 = (s + t * _NS) * _EXP_CH
            pltpu.sync_copy(acc_sh.at[pl.ds(r0, _EXP_CH)], stage_v)
            pltpu.sync_copy(
                stage_v, out_hbm.at[pl.ds(r0, _EXP_CH), pl.ds(col0, _HALF)])
            return carry

        lax.fori_loop(0, n_t, export, 0)

    return k(items, idx3d, idx_tail)


def kernel(x, idxs, n_bins, W1, b1, W2, b2):
    idx32 = jnp.minimum(idxs, N_BINS - 1).astype(jnp.int32)
    idx2d = idx32.reshape(_NS, _ROWS_PER_SUB)
    idx3d = idx2d[:, :_NCHUNK * _CH].reshape(_NS, _NCHUNK, _CH)
    idx_tail = idx2d[:, _NCHUNK * _CH:]
    items = _mlp(x, W1, b1, W2, b2)
    return _segsum(items, idx3d, idx_tail)
